# baseline (speedup 1.0000x reference)
import functools
import jax
import jax.numpy as jnp
from jax import lax
from jax.experimental import pallas as pl
from jax.experimental.pallas import tpu as pltpu
from jax.experimental.pallas import tpu_sc as plsc

_BS = 2048
_NW = 32
_CH = 8
_UNROLL = 8


def _tc_body(x_ref, emb_ref, out_ref):
    out_ref[...] = x_ref[...] + emb_ref[...]


def _tc_part(x, emb):
    B, S, D = x.shape
    grid = (S // _BS, B)
    return pl.pallas_call(
        _tc_body,
        grid=grid,
        in_specs=[
            pl.BlockSpec((1, _BS, D), lambda i, j: (j, i, 0)),
            pl.BlockSpec((_BS, D), lambda i, j: (i, 0)),
        ],
        out_specs=pl.BlockSpec((1, _BS, D), lambda i, j: (j, i, 0)),
        out_shape=jax.ShapeDtypeStruct((B, S, D), x.dtype),
    )(x, emb)


def _sc_part(xf, ef, S, D):
    rows_per_w = S // _NW
    n_chunks = rows_per_w // _CH
    chd = _CH * D
    mesh = plsc.VectorSubcoreMesh(core_axis_name="c", subcore_axis_name="s")
    vbuf = pltpu.VMEM((chd,), jnp.float32)

    @functools.partial(
        pl.kernel,
        out_type=jax.ShapeDtypeStruct((S * D,), jnp.float32),
        mesh=mesh,
        scratch_types=[vbuf] * 4 + [pltpu.SemaphoreType.DMA] * 4,
    )
    def sc_add(x_hbm, e_hbm, out_hbm, e0, e1, x0, x1, si0, si1, so0, so1):
        wid = lax.axis_index("s") * 2 + lax.axis_index("c")
        base = wid * (rows_per_w * D)
        ebuf = [e0, e1]
        xbuf = [x0, x1]
        sin = [si0, si1]
        sout = [so0, so1]

        def fire_in(c, p):
            off = base + c * chd
            return [
                pltpu.async_copy(e_hbm.at[pl.ds(off, chd)], ebuf[p], sin[p]),
                pltpu.async_copy(x_hbm.at[pl.ds(off, chd)], xbuf[p], sin[p]),
            ]

        def fire_out(c, p):
            off = base + c * chd
            return [pltpu.async_copy(xbuf[p], out_hbm.at[pl.ds(off, chd)],
                                     sout[p])]

        def compute(p):
            ev = ebuf[p]
            xv = xbuf[p]

            def body(i, carry):
                for k in range(_UNROLL):
                    o = (i * _UNROLL + k) * 16
                    xv[pl.ds(o, 16)] = xv[pl.ds(o, 16)] + ev[pl.ds(o, 16)]
                return carry

            lax.fori_loop(0, chd // (16 * _UNROLL), body, 0)

        pending_in = [None, None]
        pending_out = [None, None]
        pending_in[0] = fire_in(0, 0)
        for c in range(n_chunks):
            p = c % 2
            q = 1 - p
            if pending_out[q] is not None:
                for h in pending_out[q]:
                    h.wait()
                pending_out[q] = None
            if c + 1 < n_chunks:
                pending_in[q] = fire_in(c + 1, q)
            for h in pending_in[p]:
                h.wait()
            compute(p)
            pending_out[p] = fire_out(c, p)
        for p in (0, 1):
            if pending_out[p] is not None:
                for h in pending_out[p]:
                    h.wait()

    return sc_add(xf, ef)


def kernel(x, emb):
    B, S, D = x.shape
    e = emb[:S]
    out_tc = _tc_part(x[: B - 1], e)
    out_sc = _sc_part(x[B - 1].reshape(S * D), e.reshape(S * D), S, D)
    return jnp.concatenate([out_tc, out_sc.reshape(1, S, D)], axis=0)


# speedup vs baseline: 4.2986x; 4.2986x over previous
import jax
import jax.numpy as jnp
from jax.experimental import pallas as pl
from jax.experimental.pallas import tpu as pltpu

_BS = 2048


def _add_kernel(x_ref, emb_ref, out_ref):
    out_ref[...] = x_ref[...] + emb_ref[...]


def kernel(x, emb):
    B, S, D = x.shape
    grid = (S // _BS, B)
    return pl.pallas_call(
        _add_kernel,
        grid=grid,
        in_specs=[
            pl.BlockSpec((1, _BS, D), lambda i, j: (j, i, 0)),
            pl.BlockSpec((_BS, D), lambda i, j: (i, 0)),
        ],
        out_specs=pl.BlockSpec((1, _BS, D), lambda i, j: (j, i, 0)),
        out_shape=jax.ShapeDtypeStruct((B, S, D), x.dtype),
        compiler_params=pltpu.CompilerParams(
            dimension_semantics=("arbitrary", "arbitrary"),
        ),
    )(x, emb[:S])


# speedup vs baseline: 4.3022x; 1.0008x over previous
import jax
from jax.experimental import pallas as pl
from jax.experimental.pallas import tpu as pltpu

_BS = 2048


def _add_kernel(x_ref, emb_ref, out_ref):
    out_ref[...] = x_ref[...] + emb_ref[...]


def kernel(x, emb):
    B, S, D = x.shape
    grid = (S // _BS, B)
    return pl.pallas_call(
        _add_kernel,
        grid=grid,
        in_specs=[
            pl.BlockSpec((1, _BS, D), lambda i, j: (j, i, 0)),
            pl.BlockSpec((_BS, D), lambda i, j: (i, 0)),
        ],
        out_specs=pl.BlockSpec((1, _BS, D), lambda i, j: (j, i, 0)),
        out_shape=jax.ShapeDtypeStruct((B, S, D), x.dtype),
        compiler_params=pltpu.CompilerParams(
            dimension_semantics=("arbitrary", "arbitrary"),
        ),
    )(x, emb[:S])
